# baseline (speedup 1.0000x reference)
import jax
import jax.numpy as jnp
from jax.experimental import pallas as pl

BATCH = 16384
HIST = 200
BLK = 512


def _score_body(scores_ref, l_ref, y_ref, out_ref):
    scores = scores_ref[...]
    lblk = l_ref[...]
    yblk = y_ref[...]
    mask = (lblk > 0).astype(jnp.float32)
    e = jnp.exp(scores) * mask
    denom = jnp.sum(e, axis=1, keepdims=True)
    k_iota = jax.lax.broadcasted_iota(jnp.int32, e.shape, 1)
    num = jnp.sum(jnp.where(k_iota == yblk, e, 0.0), axis=1, keepdims=True)
    out_ref[...] = num / denom


def kernel(x, y, l, W):
    embx = jnp.take(W, x, axis=0)
    emby = jnp.take(W, l.reshape(-1), axis=0).reshape(l.shape + (W.shape[1],))
    scores = jnp.einsum('ij,ikj->ik', embx, emby)
    y2 = y.reshape(-1, 1).astype(jnp.int32)
    out = pl.pallas_call(
        _score_body,
        grid=(BATCH // BLK,),
        in_specs=[
            pl.BlockSpec((BLK, HIST), lambda i: (i, 0)),
            pl.BlockSpec((BLK, HIST), lambda i: (i, 0)),
            pl.BlockSpec((BLK, 1), lambda i: (i, 0)),
        ],
        out_specs=pl.BlockSpec((BLK, 1), lambda i: (i, 0)),
        out_shape=jax.ShapeDtypeStruct((BATCH, 1), jnp.float32),
    )(scores, l, y2)
    return out.reshape(-1)


# speedup vs baseline: 25.8439x; 25.8439x over previous
import dataclasses
import functools

import jax
import jax.numpy as jnp
from jax import lax
from jax.experimental import pallas as pl
from jax.experimental.pallas import tpu as pltpu
from jax.experimental.pallas import tpu_sc as plsc

BATCH = 16384
HIST = 200
DPAD = 8
NW = 32
PER_W = BATCH // NW
G = 16
NGROUPS = PER_W // G
ROWS = G * HIST


def _sc_body(w_hbm, lflat_hbm, x_hbm, y_hbm, out_hbm,
             xv, exall, yv, lbuf, ey, outb, sem):
    cid = lax.axis_index("c")
    sid = lax.axis_index("s")
    wid = sid * 2 + cid
    wbase = wid * PER_W

    pltpu.sync_copy(x_hbm.at[pl.ds(wbase, PER_W)], xv)
    pltpu.sync_copy(y_hbm.at[pl.ds(wbase, PER_W)], yv)
    pltpu.async_copy(w_hbm.at[xv], exall, sem).wait()

    iota = lax.iota(jnp.int32, G)
    rowbase = iota * HIST

    @pl.loop(0, NGROUPS)
    def _group(g):
        pltpu.sync_copy(lflat_hbm.at[pl.ds((wbase + g * G) * HIST, ROWS)], lbuf)
        pltpu.async_copy(w_hbm.at[lbuf], ey, sem).wait()

        gxrow = iota + g * G
        exd = [plsc.load_gather(exall, [gxrow, jnp.full((G,), d, jnp.int32)])
               for d in range(5)]
        ygrp = yv[pl.ds(g * G, G)]

        def step(k, carry):
            denom, numer = carry
            rowv = rowbase + k
            sc = exd[0] * plsc.load_gather(ey, [rowv, jnp.full((G,), 0, jnp.int32)])
            for d in range(1, 5):
                sc = sc + exd[d] * plsc.load_gather(
                    ey, [rowv, jnp.full((G,), d, jnp.int32)])
            lvals = plsc.load_gather(lbuf, [rowv])
            masked = jnp.where(lvals > 0, jnp.exp(sc), 0.0)
            denom = denom + masked
            numer = jnp.where(ygrp == k, masked, numer)
            return denom, numer

        zeros = jnp.zeros((G,), jnp.float32)
        denom, numer = lax.fori_loop(0, HIST, step, (zeros, zeros))
        outb[pl.ds(g * G, G)] = numer / denom

    pltpu.sync_copy(outb, out_hbm.at[pl.ds(wbase, PER_W)])


def kernel(x, y, l, W):
    w8 = jnp.pad(W, ((0, 0), (0, DPAD - W.shape[1])))
    lflat = l.reshape(-1).astype(jnp.int32)
    mesh = plsc.VectorSubcoreMesh(core_axis_name="c", subcore_axis_name="s")
    cp = pltpu.CompilerParams()
    for fld, val in (("needs_layout_passes", False),
                     ("use_tc_tiling_on_sc", False)):
        if fld in pltpu.CompilerParams.__dataclass_fields__:
            cp = dataclasses.replace(cp, **{fld: val})
    run = pl.kernel(
        _sc_body,
        out_type=jax.ShapeDtypeStruct((BATCH,), jnp.float32),
        mesh=mesh,
        scratch_types=[
            pltpu.VMEM((PER_W,), jnp.int32),
            pltpu.VMEM((PER_W, DPAD), jnp.float32),
            pltpu.VMEM((PER_W,), jnp.int32),
            pltpu.VMEM((ROWS,), jnp.int32),
            pltpu.VMEM((ROWS, DPAD), jnp.float32),
            pltpu.VMEM((PER_W,), jnp.float32),
            pltpu.SemaphoreType.DMA,
        ],
        compiler_params=cp,
    )
    return run(w8, lflat, x.astype(jnp.int32), y.astype(jnp.int32))
